# baseline (speedup 1.0000x reference)
import jax
import jax.numpy as jnp
from jax import lax
from jax.experimental import pallas as pl
from jax.experimental.pallas import tpu as pltpu

_B = 2048
_D = 1024
_E = 8
_KEEP = 6

_BAND_LO, _BAND_HI = 752, 1616
_RUNS = ((752, 776), (1184, 1200), (1344, 1376))
_SHIFT = 64


def _dot(xb, W):
    return lax.dot_general(xb, W, (((1,), (1,)), ((), ())),
                           preferred_element_type=jnp.float32)


def _matmul_body(x_ref, We_ref, o_ref):
    o_ref[0] = _dot(x_ref[...], We_ref[0])


def _phase1(x):
    def call(xs, We):
        return pl.pallas_call(
            _matmul_body,
            grid=(_E,),
            in_specs=[
                pl.BlockSpec((_B, _D), lambda e: (0, 0)),
                pl.BlockSpec((1, _D, _D), lambda e: (e, 0, 0)),
            ],
            out_specs=pl.BlockSpec((1, _B, _D), lambda e: (e, 0, 0)),
            out_shape=jax.ShapeDtypeStruct((_E, _B, _D), jnp.float32),
            compiler_params=pltpu.CompilerParams(
                dimension_semantics=("parallel",)),
        )(xs, We)
    return call


def _routing_logic(acc_ref, en_ref, g_ref, out_ref):
    g = [g_ref[i:i + 1, :] for i in range(_E)]
    en = [en_ref[i, :, :] for i in range(_E)]

    rk = []
    for i in range(_E):
        acc = None
        for j in range(_E):
            if j == i:
                continue
            term = (g[j] >= g[i]) if j < i else (g[j] > g[i])
            t = term.astype(jnp.int32)
            acc = t if acc is None else acc + t
        rk.append(acc)

    EN, WV, IDX = [], [], []
    for k in range(_E):
        enk = jnp.zeros_like(en[0])
        wvk = jnp.zeros_like(g[0])
        idk = jnp.zeros_like(rk[0])
        for i in range(_E):
            hit = rk[i] == k
            enk = jnp.where(hit, en[i], enk)
            wvk = jnp.where(hit, g[i], wvk)
            idk = jnp.where(hit, i, idk)
        EN.append(enk)
        WV.append(wvk)
        IDX.append(idk)

    kept = []
    for k in range(_E):
        acc = None
        for j in range(_E):
            if j == k:
                continue
            term = (EN[j] <= EN[k]) if j < k else (EN[j] < EN[k])
            t = term.astype(jnp.int32)
            acc = t if acc is None else acc + t
        kept.append(acc < _KEEP)

    t = []
    for k in range(_E):
        tk = jnp.zeros_like(IDX[0])
        for p in range(_E):
            tk = jnp.where(IDX[k] == p, IDX[p], tk)
        t.append(tk)

    zero = jnp.zeros_like(g[0])
    out = None
    for i in range(_E):
        c = zero
        for k in range(_E):
            c = c + jnp.where(kept[k] & (t[k] == i), WV[k], zero)
        term = c.T * acc_ref[i]
        out = term if out is None else out + term
    out_ref[...] = out


def _gate_and_en0(x_ref, Wg_ref, bg_ref, g_ref, en_ref):
    xb = x_ref[...]
    logits = lax.dot_general(Wg_ref[...], xb, (((0,), (1,)), ((), ())),
                             preferred_element_type=jnp.float32)
    logits = logits + bg_ref[...]
    m = jnp.max(logits, axis=0, keepdims=True)
    ex = jnp.exp(logits - m)
    g_ref[...] = ex / jnp.sum(ex, axis=0, keepdims=True)
    ones = jnp.ones_like(xb)
    en_ref[0] = jnp.log(jnp.sum(jnp.exp(ones), axis=-1, keepdims=True).T)


def _energy_into(en_ref, e, r, o):
    cos = (r * o) / (jnp.abs(r) * jnp.abs(o) + 1e-08)
    en_ref[e] = jnp.log(jnp.sum(jnp.exp(cos), axis=-1, keepdims=True)).T


def _mm_region_body(x_ref, Wg_ref, bg_ref, We_ref, be_ref, out_ref,
                    acc_ref, en_ref, g_ref):
    e = pl.program_id(1)
    xb = x_ref[...]
    o = _dot(xb, We_ref[0]) + be_ref[0]
    acc_ref[e] = o

    @pl.when(e == 0)
    def _g():
        _gate_and_en0(x_ref, Wg_ref, bg_ref, g_ref, en_ref)

    @pl.when(e > 0)
    def _e():
        _energy_into(en_ref, e, acc_ref[0], o)

    @pl.when(e == _E - 1)
    def _c():
        _routing_logic(acc_ref, en_ref, g_ref, out_ref)


def _band_region_body(x_ref, Wg_ref, bg_ref, ao1_ref, ao2_ref, be_ref, out_ref,
                      acc_ref, en_ref, g_ref, bm):
    e = pl.program_id(1)
    rb = pl.program_id(0)

    row = (_BAND_LO + rb * bm
           + lax.broadcasted_iota(jnp.int32, (bm, 1), 0))
    runmask = None
    for lo, hi in _RUNS:
        m = (row >= lo) & (row < hi)
        runmask = m if runmask is None else (runmask | m)
    o = jnp.where(runmask, ao2_ref[0], ao1_ref[0]) + be_ref[0]
    acc_ref[e] = o

    @pl.when(e == 0)
    def _g():
        _gate_and_en0(x_ref, Wg_ref, bg_ref, g_ref, en_ref)

    @pl.when(e > 0)
    def _e():
        _energy_into(en_ref, e, acc_ref[0], o)

    @pl.when(e == _E - 1)
    def _c():
        _routing_logic(acc_ref, en_ref, g_ref, out_ref)


def _scratch(bm):
    return [
        pltpu.VMEM((_E, bm, _D), jnp.float32),
        pltpu.VMEM((_E, 1, bm), jnp.float32),
        pltpu.VMEM((_E, bm), jnp.float32),
    ]


def _mm_region(xs, Wg, bg2, We, be3, bm):
    rows = xs.shape[0]
    return pl.pallas_call(
        _mm_region_body,
        grid=(rows // bm, _E),
        in_specs=[
            pl.BlockSpec((bm, _D), lambda rb, e: (rb, 0)),
            pl.BlockSpec((_D, _E), lambda rb, e: (0, 0)),
            pl.BlockSpec((_E, 1), lambda rb, e: (0, 0)),
            pl.BlockSpec((1, _D, _D), lambda rb, e: (e, 0, 0)),
            pl.BlockSpec((1, 1, _D), lambda rb, e: (e, 0, 0)),
        ],
        out_specs=pl.BlockSpec((bm, _D), lambda rb, e: (rb, 0)),
        out_shape=jax.ShapeDtypeStruct((rows, _D), jnp.float32),
        scratch_shapes=_scratch(bm),
        compiler_params=pltpu.CompilerParams(
            dimension_semantics=("parallel", "arbitrary")),
    )(xs, Wg, bg2, We, be3)


def _band_region(xs, Wg, bg2, ao1, ao2, be3, bm):
    rows = xs.shape[0]
    import functools
    body = functools.partial(_band_region_body, bm=bm)
    return pl.pallas_call(
        body,
        grid=(rows // bm, _E),
        in_specs=[
            pl.BlockSpec((bm, _D), lambda rb, e: (rb, 0)),
            pl.BlockSpec((_D, _E), lambda rb, e: (0, 0)),
            pl.BlockSpec((_E, 1), lambda rb, e: (0, 0)),
            pl.BlockSpec((1, bm, _D), lambda rb, e: (e, rb, 0)),
            pl.BlockSpec((1, bm, _D), lambda rb, e: (e, rb, 0)),
            pl.BlockSpec((1, 1, _D), lambda rb, e: (e, 0, 0)),
        ],
        out_specs=pl.BlockSpec((bm, _D), lambda rb, e: (rb, 0)),
        out_shape=jax.ShapeDtypeStruct((rows, _D), jnp.float32),
        scratch_shapes=_scratch(bm),
        compiler_params=pltpu.CompilerParams(
            dimension_semantics=("parallel", "arbitrary")),
    )(xs, Wg, bg2, ao1, ao2, be3)


@jax.jit
def kernel(x, Wg, bg, We, be):
    bg2 = bg.reshape(_E, 1)
    be3 = be.reshape(_E, 1, _D)

    mm = _phase1(x)
    ao_m = mm(x, We)
    ao_m2 = mm(jnp.roll(x, _SHIFT, axis=0), We)
    ao1 = ao_m[:, _BAND_LO:_BAND_HI, :]
    ao2 = ao_m2[:, _BAND_LO + _SHIFT:_BAND_HI + _SHIFT, :]

    out_a = _mm_region(x[:_BAND_LO], Wg, bg2, We, be3, 376)
    out_b = _band_region(x[_BAND_LO:_BAND_HI], Wg, bg2, ao1, ao2, be3, 432)
    out_c = _mm_region(x[_BAND_HI:], Wg, bg2, We, be3, 432)
    return jnp.concatenate([out_a, out_b, out_c], axis=0)
